# Initial kernel scaffold; baseline (speedup 1.0000x reference)
import jax
import jax.numpy as jnp
from jax.experimental import pallas as pl


def kernel(x, seg, word_table, pos_table, type_table, gamma, beta):
    raise NotImplementedError("write your pallas kernel here")


# speedup vs baseline: 2.5645x; 2.5645x over previous
import math

import jax
import jax.numpy as jnp
from jax import lax
from jax.experimental import pallas as pl
from jax.experimental.pallas import tpu as pltpu
from jax.experimental.pallas import tpu_sc as plsc

_NC = 2
_NS = 16
_NW = _NC * _NS

_BATCH = 4
_SEQ = 2048
_D = 128
_TOKENS = _BATCH * _SEQ
_B_PER_W = _TOKENS // _NW
_CHUNKS = _B_PER_W // 128


def _sc_gather_body(table_hbm, idx_hbm, out_hbm, idx_v, rows_v, sem):
    wid = lax.axis_index("s") * _NC + lax.axis_index("c")
    base = wid * _B_PER_W
    pltpu.sync_copy(idx_hbm.at[wid], idx_v)
    copies = []
    for j in range(_CHUNKS):
        copies.append(
            pltpu.async_copy(
                table_hbm.at[idx_v.at[j]],
                rows_v.at[pl.ds(j * 128, 128)],
                sem,
            )
        )
    for c in copies:
        c.wait()
    pltpu.sync_copy(rows_v, out_hbm.at[pl.ds(base, _B_PER_W)])


def _sc_gather(word_table, idx):
    mesh = plsc.VectorSubcoreMesh(
        core_axis_name="c", subcore_axis_name="s",
        num_cores=_NC, num_subcores=_NS,
    )
    return pl.kernel(
        _sc_gather_body,
        out_type=jax.ShapeDtypeStruct((_TOKENS, _D), jnp.float32),
        mesh=mesh,
        scratch_types=[
            pltpu.VMEM((_CHUNKS, 128), jnp.int32),
            pltpu.VMEM((_B_PER_W, _D), jnp.float32),
            pltpu.SemaphoreType.DMA,
        ],
    )(word_table, idx)


_SCALE = 1.0 / math.sqrt(2.0 * math.log(_D))


def _tc_norm_body(gw_ref, seg_ref, pos_ref, type_ref, gamma_ref, beta_ref,
                  out_ref):
    gw = gw_ref[...]
    segf = seg_ref[...].astype(jnp.float32)
    pos = pos_ref[...]
    t0 = type_ref[0:1, :]
    t1 = type_ref[1:2, :]
    gamma = gamma_ref[0:1, :]
    beta = beta_ref[0:1, :]

    emb = gw + pos[None, :, :] + t0[None, :, :] \
        + segf[:, :, None] * (t1 - t0)[None, :, :]
    mean = jnp.mean(emb, axis=-1, keepdims=True)
    y = emb - mean
    xmin = jnp.min(y, axis=1, keepdims=True)
    xmax = jnp.max(y, axis=1, keepdims=True)
    out = y / ((xmax - xmin) * _SCALE)
    out_ref[...] = out * gamma[None, :, :] + beta[None, :, :]


def _tc_norm(gathered, seg, pos_table, type_table, gamma, beta):
    return pl.pallas_call(
        _tc_norm_body,
        out_shape=jax.ShapeDtypeStruct((_BATCH, _SEQ, _D), jnp.float32),
    )(gathered, seg, pos_table, type_table,
      gamma.reshape(1, _D), beta.reshape(1, _D))


def kernel(x, seg, word_table, pos_table, type_table, gamma, beta):
    idx = x.astype(jnp.int32).reshape(_NW, _CHUNKS, 128)
    gathered = _sc_gather(word_table, idx)
    return _tc_norm(gathered.reshape(_BATCH, _SEQ, _D), seg,
                    pos_table, type_table, gamma, beta)


# speedup vs baseline: 2.5985x; 1.0133x over previous
import math

import jax
import jax.numpy as jnp
from jax import lax
from jax.experimental import pallas as pl
from jax.experimental.pallas import tpu as pltpu
from jax.experimental.pallas import tpu_sc as plsc

_NC = 2
_NS = 16
_NW = _NC * _NS

_BATCH = 4
_SEQ = 2048
_D = 128
_TOKENS = _BATCH * _SEQ
_B_PER_W = _TOKENS // _NW
_CHUNKS = _B_PER_W // 128


def _sc_gather_body(table_hbm, idx_hbm, out_hbm, idx_v, rows_v, sem):
    wid = lax.axis_index("s") * _NC + lax.axis_index("c")
    base = wid * _B_PER_W
    pltpu.sync_copy(idx_hbm.at[wid], idx_v)
    copies = []
    for j in range(_CHUNKS):
        copies.append(
            pltpu.async_copy(
                table_hbm.at[idx_v.at[j]],
                rows_v.at[pl.ds(j * 128, 128)],
                sem,
            )
        )
    for c in copies:
        c.wait()
    pltpu.sync_copy(rows_v, out_hbm.at[pl.ds(base, _B_PER_W)])


def _sc_gather(word_table, idx):
    mesh = plsc.VectorSubcoreMesh(
        core_axis_name="c", subcore_axis_name="s",
        num_cores=_NC, num_subcores=_NS,
    )
    return pl.kernel(
        _sc_gather_body,
        out_type=jax.ShapeDtypeStruct((_TOKENS, _D), jnp.float32),
        mesh=mesh,
        scratch_types=[
            pltpu.VMEM((_CHUNKS, 128), jnp.int32),
            pltpu.VMEM((_B_PER_W, _D), jnp.float32),
            pltpu.SemaphoreType.DMA,
        ],
    )(word_table, idx)


_SCALE = 1.0 / math.sqrt(2.0 * math.log(_D))


def _tc_norm_body(gw_ref, seg_ref, pos_ref, type_ref, gamma_ref, beta_ref,
                  out_ref):
    gw = gw_ref[0]
    segf = seg_ref[0, 0].astype(jnp.float32)
    pos = pos_ref[...]
    t0 = type_ref[0:1, :]
    t1 = type_ref[1:2, :]
    gamma = gamma_ref[0:1, :]
    beta = beta_ref[0:1, :]

    emb = gw + pos + t0 + segf[:, None] * (t1 - t0)
    mean = jnp.mean(emb, axis=-1, keepdims=True)
    y = emb - mean
    xmin = jnp.min(y, axis=0, keepdims=True)
    xmax = jnp.max(y, axis=0, keepdims=True)
    out = y / ((xmax - xmin) * _SCALE)
    out_ref[0] = out * gamma + beta


def _tc_norm(gathered, seg, pos_table, type_table, gamma, beta):
    return pl.pallas_call(
        _tc_norm_body,
        grid=(_BATCH,),
        in_specs=[
            pl.BlockSpec((1, _SEQ, _D), lambda b: (b, 0, 0)),
            pl.BlockSpec((1, 1, _SEQ), lambda b: (b, 0, 0)),
            pl.BlockSpec((_SEQ, _D), lambda b: (0, 0)),
            pl.BlockSpec((2, _D), lambda b: (0, 0)),
            pl.BlockSpec((1, _D), lambda b: (0, 0)),
            pl.BlockSpec((1, _D), lambda b: (0, 0)),
        ],
        out_specs=pl.BlockSpec((1, _SEQ, _D), lambda b: (b, 0, 0)),
        out_shape=jax.ShapeDtypeStruct((_BATCH, _SEQ, _D), jnp.float32),
    )(gathered, seg.reshape(_BATCH, 1, _SEQ), pos_table, type_table,
      gamma.reshape(1, _D), beta.reshape(1, _D))


def kernel(x, seg, word_table, pos_table, type_table, gamma, beta):
    idx = x.astype(jnp.int32).reshape(_NW, _CHUNKS, 128)
    gathered = _sc_gather(word_table, idx)
    return _tc_norm(gathered.reshape(_BATCH, _SEQ, _D), seg.astype(jnp.int32),
                    pos_table, type_table, gamma, beta)
